# Initial kernel scaffold; baseline (speedup 1.0000x reference)
import jax
import jax.numpy as jnp
from jax.experimental import pallas as pl


def kernel(x, wide_w, wide_b, emb, W1, b1, g1, be1, W2, b2, g2, be2, W3, b3):
    raise NotImplementedError("write your pallas kernel here")


# speedup vs baseline: 19.6876x; 19.6876x over previous
import functools

import jax
import jax.numpy as jnp
from jax.experimental import pallas as pl
from jax.experimental.pallas import tpu as pltpu

B = 4096
F = 100
D = 64
H = 512
BB = 512
NB = B // BB
VOCAB_PAD = 128
EPS = 1e-5


def _wnd_kernel(xT_ref, embT_ref, W1_ref, W2_ref, W3_ref, wideW_ref,
                b1_ref, g1_ref, be1_ref, b2_ref, g2_ref, be2_ref, c3_ref,
                out_ref, dT, h1T, h2T, s1, q1, s2, q2):
    phase = pl.program_id(0)
    j = pl.program_id(1)

    @pl.when(phase < 2)
    def _pfill():
        out_ref[...] = jnp.zeros((1, 1, BB), jnp.float32)

    @pl.when(phase == 0)
    def _p0():
        embT = embT_ref[...]
        for f in range(F):
            idx = xT_ref[pl.ds(f, 1), :]
            idxb = jnp.broadcast_to(idx, (D, BB))
            g = jnp.take_along_axis(embT, idxb, axis=1)
            dT[pl.ds(f * D, D), :] = g.astype(jnp.bfloat16)
        h1 = jnp.dot(W1_ref[...], dT[...],
                     preferred_element_type=jnp.float32)
        h1 = h1 + b1_ref[...]
        h1T[j] = h1
        bs = jnp.sum(h1, axis=1, keepdims=True)
        bq = jnp.sum(h1 * h1, axis=1, keepdims=True)

        @pl.when(j == 0)
        def _():
            s1[...] = bs
            q1[...] = bq

        @pl.when(j > 0)
        def _():
            s1[...] += bs
            q1[...] += bq

    @pl.when(phase == 1)
    def _p1():
        mu = s1[...] * (1.0 / B)
        var = q1[...] * (1.0 / B) - mu * mu
        rs = jax.lax.rsqrt(var + EPS)
        a = g1_ref[...] * rs
        c = be1_ref[...] - mu * a
        nh = jnp.maximum(h1T[j] * a + c, 0.0).astype(jnp.bfloat16)
        h2 = jnp.dot(W2_ref[...], nh,
                     preferred_element_type=jnp.float32) + b2_ref[...]
        h2T[j] = h2
        bs = jnp.sum(h2, axis=1, keepdims=True)
        bq = jnp.sum(h2 * h2, axis=1, keepdims=True)

        @pl.when(j == 0)
        def _():
            s2[...] = bs
            q2[...] = bq

        @pl.when(j > 0)
        def _():
            s2[...] += bs
            q2[...] += bq

    @pl.when(phase == 2)
    def _p2():
        mu = s2[...] * (1.0 / B)
        var = q2[...] * (1.0 / B) - mu * mu
        rs = jax.lax.rsqrt(var + EPS)
        a = g2_ref[...] * rs
        c = be2_ref[...] - mu * a
        nh = jnp.maximum(h2T[j] * a + c, 0.0)
        logit = jnp.sum(nh * W3_ref[...], axis=0, keepdims=True)
        xf = xT_ref[...].astype(jnp.float32)
        wide = jnp.sum(xf * wideW_ref[...], axis=0, keepdims=True)
        z = logit + wide + c3_ref[...]
        out_ref[...] = jax.nn.sigmoid(z).reshape(1, 1, BB)


@functools.partial(jax.jit, static_argnames=())
def kernel(x, wide_w, wide_b, emb, W1, b1, g1, be1, W2, b2, g2, be2, W3, b3):
    xT = x.astype(jnp.int32).T
    embT = jnp.zeros((D, VOCAB_PAD), jnp.float32).at[:, :F].set(emb.T)
    W1b = W1.astype(jnp.bfloat16)
    W2b = W2.astype(jnp.bfloat16)
    W3c = W3.reshape(H, 1)
    wideWc = wide_w.reshape(F, 1)
    col = lambda v: v.reshape(-1, 1)
    c3 = (b3 + wide_b).reshape(1, 1)

    full = lambda shape: pl.BlockSpec(shape, lambda p, j: (0, 0))
    grid = (3, NB)
    out = pl.pallas_call(
        _wnd_kernel,
        grid=grid,
        in_specs=[
            pl.BlockSpec((F, BB), lambda p, j: (0, j)),
            full((D, VOCAB_PAD)),
            full((H, F * D)),
            full((H, H)),
            full((H, 1)),
            full((F, 1)),
            full((H, 1)), full((H, 1)), full((H, 1)),
            full((H, 1)), full((H, 1)), full((H, 1)),
            full((1, 1)),
        ],
        out_specs=pl.BlockSpec((1, 1, BB), lambda p, j: (p, 0, j)),
        out_shape=jax.ShapeDtypeStruct((3, 1, B), jnp.float32),
        scratch_shapes=[
            pltpu.VMEM((F * D, BB), jnp.bfloat16),
            pltpu.VMEM((NB, H, BB), jnp.float32),
            pltpu.VMEM((NB, H, BB), jnp.float32),
            pltpu.VMEM((H, 1), jnp.float32),
            pltpu.VMEM((H, 1), jnp.float32),
            pltpu.VMEM((H, 1), jnp.float32),
            pltpu.VMEM((H, 1), jnp.float32),
        ],
        compiler_params=pltpu.CompilerParams(
            dimension_semantics=("arbitrary", "arbitrary"),
            vmem_limit_bytes=100 * 1024 * 1024,
        ),
    )(xT, embT, W1b, W2b, W3c, wideWc,
      col(b1), col(g1), col(be1), col(b2), col(g2), col(be2), c3)
    return out[2].reshape(B, 1)
